# Initial kernel scaffold; baseline (speedup 1.0000x reference)
import jax
import jax.numpy as jnp
from jax.experimental import pallas as pl


def kernel(x, edge_index, W1_rel, b1, W1_root, W2_rel, b2, W2_root):
    raise NotImplementedError("write your pallas kernel here")


# speedup vs baseline: 10.2387x; 10.2387x over previous
import functools

import jax
import jax.numpy as jnp
from jax import lax
from jax.experimental import pallas as pl
from jax.experimental.pallas import tpu as pltpu
from jax.experimental.pallas import tpu_sc as plsc

N = 10000
D = 128
E = 320000
H = 16
C = 10

NC = 2
NS = 16
NW = NC * NS
CHUNK = 128
NCH = 80
EPW = NCH * CHUNK
E_PAD = NW * EPW
VROWS = N + 16
ACC_ROWS = 10240
ZROWS = ACC_ROWS // NS
RPT = N // NS


def _sc_segsum_body(vals_hbm, src_hbm, dst_hbm, zeros_hbm, out_hbm,
                    src_v, dst_v, rows_v, acc_sh, sem):
    c = lax.axis_index("c")
    s = lax.axis_index("s")
    wid = c * NS + s
    pltpu.sync_copy(zeros_hbm, acc_sh.at[pl.ds(s * ZROWS, ZROWS)])
    pltpu.sync_copy(src_hbm.at[wid], src_v)
    pltpu.sync_copy(dst_hbm.at[wid], dst_v)
    plsc.subcore_barrier()

    @pl.loop(0, NCH)
    def _(j):
        pltpu.async_copy(vals_hbm.at[src_v.at[j]], rows_v, sem).wait()
        pltpu.sync_copy(rows_v, acc_sh.at[dst_v.at[j]], add=True)

    plsc.subcore_barrier()
    pltpu.sync_copy(acc_sh.at[pl.ds(s * ZROWS, ZROWS)],
                    out_hbm.at[c].at[pl.ds(s * ZROWS, ZROWS)])


@functools.cache
def _sc_segsum():
    mesh = plsc.VectorSubcoreMesh(core_axis_name="c", subcore_axis_name="s",
                                  num_cores=NC)
    return pl.kernel(
        _sc_segsum_body,
        out_type=jax.ShapeDtypeStruct((NC, ACC_ROWS, 16), jnp.float32),
        mesh=mesh,
        compiler_params=pltpu.CompilerParams(use_tc_tiling_on_sc=False),
        scratch_types=[
            pltpu.VMEM((NCH, CHUNK), jnp.int32),
            pltpu.VMEM((NCH, CHUNK), jnp.int32),
            pltpu.VMEM((CHUNK, 16), jnp.float32),
            pltpu.VMEM_SHARED((ACC_ROWS, 16), jnp.float32),
            pltpu.SemaphoreType.DMA,
        ],
    )


def _proj1_body(x_ref, w_ref, xr_ref, xroot_ref):
    p = lax.dot_general(x_ref[...], w_ref[...], (((1,), (1,)), ((), ())),
                        preferred_element_type=jnp.float32)
    xr_ref[...] = jnp.concatenate(
        [p[:, :16], jnp.zeros((VROWS - N, 16), jnp.float32)], axis=0)
    xroot_ref[...] = p[:, 16:]


_proj1 = pl.pallas_call(
    _proj1_body,
    out_shape=(jax.ShapeDtypeStruct((VROWS, 16), jnp.float32),
               jax.ShapeDtypeStruct((N, 16), jnp.float32)),
)


def _mid_body(parts_ref, xroot_ref, b1_ref, w2_ref, hr_ref, hroot_ref):
    agg = parts_ref[0, :N] + parts_ref[1, :N]
    h = jnp.maximum(agg + xroot_ref[...] + b1_ref[...], 0.0)
    p = lax.dot_general(h, w2_ref[...], (((1,), (1,)), ((), ())),
                        preferred_element_type=jnp.float32)
    hr_ref[...] = jnp.concatenate(
        [p[:, :16], jnp.zeros((VROWS - N, 16), jnp.float32)], axis=0)
    hroot_ref[...] = p[:, 16:]


_mid = pl.pallas_call(
    _mid_body,
    out_shape=(jax.ShapeDtypeStruct((VROWS, 16), jnp.float32),
               jax.ShapeDtypeStruct((N, 16), jnp.float32)),
)


def _out_body(parts_ref, hroot_ref, b2_ref, o_ref):
    o = parts_ref[0, :N] + parts_ref[1, :N] + hroot_ref[...] + b2_ref[...]
    valid = lax.broadcasted_iota(jnp.int32, (1, 16), 1) < C
    om = jnp.where(valid, o, -1e30)
    m = jnp.max(om, axis=1, keepdims=True)
    lse = m + jnp.log(jnp.sum(jnp.exp(om - m), axis=1, keepdims=True))
    o_ref[...] = (o - lse)[:, :C]


_outk = pl.pallas_call(
    _out_body,
    out_shape=jax.ShapeDtypeStruct((N, C), jnp.float32),
)


def kernel(x, edge_index, W1_rel, b1, W1_root, W2_rel, b2, W2_root):
    w1c = jnp.concatenate([W1_rel, W1_root], axis=0)
    w2c = jnp.zeros((32, H), jnp.float32)
    w2c = w2c.at[:C].set(W2_rel).at[16:16 + C].set(W2_root)
    b1r = b1.reshape(1, H)
    b2r = jnp.zeros((1, 16), jnp.float32).at[:, :C].set(b2)

    pad = jnp.full((E_PAD - E,), N, jnp.int32)
    src = jnp.concatenate([edge_index[0], pad]).reshape(NW, NCH, CHUNK)
    dst = jnp.concatenate([edge_index[1], pad]).reshape(NW, NCH, CHUNK)
    zrows = jnp.zeros((ZROWS, 16), jnp.float32)

    segsum = _sc_segsum()
    xr, xroot = _proj1(x, w1c)
    parts1 = segsum(xr, src, dst, zrows)
    hr, hroot = _mid(parts1, xroot, b1r, w2c)
    parts2 = segsum(hr, src, dst, zrows)
    return _outk(parts2, hroot, b2r)


# speedup vs baseline: 14.2545x; 1.3922x over previous
import functools

import jax
import jax.numpy as jnp
from jax import lax
from jax.experimental import pallas as pl
from jax.experimental.pallas import tpu as pltpu
from jax.experimental.pallas import tpu_sc as plsc

N = 10000
D = 128
E = 320000
H = 16
C = 10

NC = 2
NS = 16
NW = NC * NS
CHUNK = 128
NCH = 80
EPW = NCH * CHUNK
E_PAD = NW * EPW
VROWS = N + 16
ACC_ROWS = 10240
ZROWS = ACC_ROWS // NS
RPT = N // NS


G = 4
ROUNDS = NCH // G


def _sc_segsum_body(vals_hbm, src_hbm, dst_hbm, zeros_hbm, out_hbm,
                    src_v, dst_v, rows_v, acc_sh, gsA, gsB, ssA, ssB):
    c = lax.axis_index("c")
    s = lax.axis_index("s")
    wid = c * NS + s
    pltpu.sync_copy(zeros_hbm, acc_sh.at[pl.ds(s * ZROWS, ZROWS)])
    pltpu.sync_copy(src_hbm.at[wid], src_v)
    pltpu.sync_copy(dst_hbm.at[wid], dst_v)
    plsc.subcore_barrier()

    sems = (gsA, gsB, ssA, ssB)

    def start_gather(b, j):
        pltpu.async_copy(vals_hbm.at[src_v.at[j]], rows_v.at[b], sems[b])

    def wait_gather(b, j):
        pltpu.make_async_copy(vals_hbm.at[src_v.at[j]], rows_v.at[b],
                              sems[b]).wait()

    for b in range(G):
        start_gather(b, b)

    @pl.loop(0, NCH - G, step=G)
    def _(jj):
        for b in range(G):
            wait_gather(b, jj + b)
            pltpu.sync_copy(rows_v.at[b], acc_sh.at[dst_v.at[jj + b]],
                            add=True)
            start_gather(b, jj + G + b)

    for b in range(G):
        jj = NCH - G
        wait_gather(b, jj + b)
        pltpu.sync_copy(rows_v.at[b], acc_sh.at[dst_v.at[jj + b]], add=True)

    plsc.subcore_barrier()
    pltpu.sync_copy(acc_sh.at[pl.ds(s * ZROWS, ZROWS)],
                    out_hbm.at[c].at[pl.ds(s * ZROWS, ZROWS)])


@functools.cache
def _sc_segsum():
    mesh = plsc.VectorSubcoreMesh(core_axis_name="c", subcore_axis_name="s",
                                  num_cores=NC)
    return pl.kernel(
        _sc_segsum_body,
        out_type=jax.ShapeDtypeStruct((NC, ACC_ROWS, 16), jnp.float32),
        mesh=mesh,
        compiler_params=pltpu.CompilerParams(use_tc_tiling_on_sc=False),
        scratch_types=[
            pltpu.VMEM((NCH, CHUNK), jnp.int32),
            pltpu.VMEM((NCH, CHUNK), jnp.int32),
            pltpu.VMEM((G, CHUNK, 16), jnp.float32),
            pltpu.VMEM_SHARED((ACC_ROWS, 16), jnp.float32),
            pltpu.SemaphoreType.DMA,
            pltpu.SemaphoreType.DMA,
            pltpu.SemaphoreType.DMA,
            pltpu.SemaphoreType.DMA,
        ],
    )


def _proj1_body(x_ref, w_ref, xr_ref, xroot_ref):
    p = lax.dot_general(x_ref[...], w_ref[...], (((1,), (1,)), ((), ())),
                        preferred_element_type=jnp.float32)
    xr_ref[...] = jnp.concatenate(
        [p[:, :16], jnp.zeros((VROWS - N, 16), jnp.float32)], axis=0)
    xroot_ref[...] = p[:, 16:]


_proj1 = pl.pallas_call(
    _proj1_body,
    out_shape=(jax.ShapeDtypeStruct((VROWS, 16), jnp.float32),
               jax.ShapeDtypeStruct((N, 16), jnp.float32)),
)


def _mid_body(parts_ref, xroot_ref, b1_ref, w2_ref, hr_ref, hroot_ref):
    agg = parts_ref[0, :N] + parts_ref[1, :N]
    h = jnp.maximum(agg + xroot_ref[...] + b1_ref[...], 0.0)
    p = lax.dot_general(h, w2_ref[...], (((1,), (1,)), ((), ())),
                        preferred_element_type=jnp.float32)
    hr_ref[...] = jnp.concatenate(
        [p[:, :16], jnp.zeros((VROWS - N, 16), jnp.float32)], axis=0)
    hroot_ref[...] = p[:, 16:]


_mid = pl.pallas_call(
    _mid_body,
    out_shape=(jax.ShapeDtypeStruct((VROWS, 16), jnp.float32),
               jax.ShapeDtypeStruct((N, 16), jnp.float32)),
)


def _out_body(parts_ref, hroot_ref, b2_ref, o_ref):
    o = parts_ref[0, :N] + parts_ref[1, :N] + hroot_ref[...] + b2_ref[...]
    valid = lax.broadcasted_iota(jnp.int32, (1, 16), 1) < C
    om = jnp.where(valid, o, -1e30)
    m = jnp.max(om, axis=1, keepdims=True)
    lse = m + jnp.log(jnp.sum(jnp.exp(om - m), axis=1, keepdims=True))
    o_ref[...] = (o - lse)[:, :C]


_outk = pl.pallas_call(
    _out_body,
    out_shape=jax.ShapeDtypeStruct((N, C), jnp.float32),
)


def kernel(x, edge_index, W1_rel, b1, W1_root, W2_rel, b2, W2_root):
    w1c = jnp.concatenate([W1_rel, W1_root], axis=0)
    w2c = jnp.zeros((32, H), jnp.float32)
    w2c = w2c.at[:C].set(W2_rel).at[16:16 + C].set(W2_root)
    b1r = b1.reshape(1, H)
    b2r = jnp.zeros((1, 16), jnp.float32).at[:, :C].set(b2)

    pad = jnp.full((E_PAD - E,), N, jnp.int32)
    src = jnp.concatenate([edge_index[0], pad]).reshape(NW, NCH, CHUNK)
    dst = jnp.concatenate([edge_index[1], pad]).reshape(NW, NCH, CHUNK)
    zrows = jnp.zeros((ZROWS, 16), jnp.float32)

    segsum = _sc_segsum()
    xr, xroot = _proj1(x, w1c)
    parts1 = segsum(xr, src, dst, zrows)
    hr, hroot = _mid(parts1, xroot, b1r, w2c)
    parts2 = segsum(hr, src, dst, zrows)
    return _outk(parts2, hroot, b2r)


# speedup vs baseline: 18.1598x; 1.2740x over previous
import functools

import jax
import jax.numpy as jnp
from jax import lax
from jax.experimental import pallas as pl
from jax.experimental.pallas import tpu as pltpu
from jax.experimental.pallas import tpu_sc as plsc

N = 10000
D = 128
E = 320000
H = 16
C = 10

NC = 2
NS = 16
NW = NC * NS
CHUNK = 128
NCH = 80
EPW = NCH * CHUNK
E_PAD = NW * EPW
VROWS = N + 16
ACC_ROWS = 10240
ZROWS = ACC_ROWS // NS
RPT = N // NS


G = 4
ROUNDS = NCH // G


def _sc_segsum_body(vals_hbm, src_hbm, dst_hbm, zeros_hbm, out_hbm,
                    src_v, dst_v, rows_v, acc_sh, gsA, gsB, ssA, ssB):
    c = lax.axis_index("c")
    s = lax.axis_index("s")
    wid = c * NS + s
    pltpu.sync_copy(zeros_hbm, acc_sh.at[pl.ds(s * ZROWS, ZROWS)])
    pltpu.sync_copy(src_hbm.at[wid], src_v)
    pltpu.sync_copy(dst_hbm.at[wid], dst_v)
    plsc.subcore_barrier()

    sems = (gsA, gsB, ssA, ssB)

    def start_gather(b, j):
        pltpu.async_copy(vals_hbm.at[src_v.at[j]], rows_v.at[b], sems[b])

    def wait_gather(b, j):
        pltpu.make_async_copy(vals_hbm.at[src_v.at[j]], rows_v.at[b],
                              sems[b]).wait()

    for b in range(G):
        start_gather(b, b)

    @pl.loop(0, NCH - G, step=G)
    def _(jj):
        for b in range(G):
            wait_gather(b, jj + b)
            pltpu.sync_copy(rows_v.at[b], acc_sh.at[dst_v.at[jj + b]],
                            add=True)
            start_gather(b, jj + G + b)

    for b in range(G):
        jj = NCH - G
        wait_gather(b, jj + b)
        pltpu.sync_copy(rows_v.at[b], acc_sh.at[dst_v.at[jj + b]], add=True)

    plsc.subcore_barrier()
    pltpu.sync_copy(acc_sh.at[pl.ds(s * ZROWS, ZROWS)],
                    out_hbm.at[c].at[pl.ds(s * ZROWS, ZROWS)])


@functools.cache
def _sc_segsum():
    mesh = plsc.VectorSubcoreMesh(core_axis_name="c", subcore_axis_name="s",
                                  num_cores=NC)
    return pl.kernel(
        _sc_segsum_body,
        out_type=jax.ShapeDtypeStruct((NC, ACC_ROWS, 16), jnp.float32),
        mesh=mesh,
        compiler_params=pltpu.CompilerParams(use_tc_tiling_on_sc=False),
        scratch_types=[
            pltpu.VMEM((NCH, CHUNK), jnp.int32),
            pltpu.VMEM((NCH, CHUNK), jnp.int32),
            pltpu.VMEM((G, CHUNK, 16), jnp.float32),
            pltpu.VMEM_SHARED((ACC_ROWS, 16), jnp.float32),
            pltpu.SemaphoreType.DMA,
            pltpu.SemaphoreType.DMA,
            pltpu.SemaphoreType.DMA,
            pltpu.SemaphoreType.DMA,
        ],
    )


def _proj1_body(x_ref, w_ref, xr_ref, xroot_ref):
    p = lax.dot_general(x_ref[...], w_ref[...], (((1,), (1,)), ((), ())),
                        preferred_element_type=jnp.float32)
    xr_ref[...] = jnp.concatenate(
        [p[:, :16], jnp.zeros((VROWS - N, 16), jnp.float32)], axis=0)
    xroot_ref[...] = p[:, 16:]


_proj1 = pl.pallas_call(
    _proj1_body,
    out_shape=(jax.ShapeDtypeStruct((VROWS, 16), jnp.float32),
               jax.ShapeDtypeStruct((N, 16), jnp.float32)),
)


def _mid_body(parts_ref, xroot_ref, b1_ref, w2_ref, hr_ref, hroot_ref):
    agg = parts_ref[0, :N] + parts_ref[1, :N]
    h = jnp.maximum(agg + xroot_ref[...] + b1_ref[...], 0.0)
    p = lax.dot_general(h, w2_ref[...], (((1,), (1,)), ((), ())),
                        preferred_element_type=jnp.float32)
    hr_ref[...] = jnp.concatenate(
        [p[:, :16], jnp.zeros((VROWS - N, 16), jnp.float32)], axis=0)
    hroot_ref[...] = p[:, 16:]


_mid = pl.pallas_call(
    _mid_body,
    out_shape=(jax.ShapeDtypeStruct((VROWS, 16), jnp.float32),
               jax.ShapeDtypeStruct((N, 16), jnp.float32)),
)


def _out_body(parts_ref, hroot_ref, b2_ref, o_ref):
    o = parts_ref[0, :N] + parts_ref[1, :N] + hroot_ref[...] + b2_ref[...]
    valid = lax.broadcasted_iota(jnp.int32, (1, 16), 1) < C
    om = jnp.where(valid, o, -1e30)
    m = jnp.max(om, axis=1, keepdims=True)
    lse = m + jnp.log(jnp.sum(jnp.exp(om - m), axis=1, keepdims=True))
    o_ref[...] = (o - lse)[:, :C]


_outk = pl.pallas_call(
    _out_body,
    out_shape=jax.ShapeDtypeStruct((N, C), jnp.float32),
)


def kernel(x, edge_index, W1_rel, b1, W1_root, W2_rel, b2, W2_root):
    w1c = jnp.concatenate([W1_rel, W1_root], axis=0)
    w2c = jnp.zeros((32, H), jnp.float32)
    w2c = w2c.at[:C].set(W2_rel).at[16:16 + C].set(W2_root)
    b1r = b1.reshape(1, H)
    b2r = jnp.zeros((1, 16), jnp.float32).at[:, :C].set(b2)

    ar = jnp.arange(E_PAD - E, dtype=jnp.int32)
    pad_src = N + ar % (VROWS - N)
    pad_dst = N + ar % (ACC_ROWS - N)
    src = jnp.concatenate([edge_index[0], pad_src]).reshape(NW, NCH, CHUNK)
    dst = jnp.concatenate([edge_index[1], pad_dst]).reshape(NW, NCH, CHUNK)
    zrows = jnp.zeros((ZROWS, 16), jnp.float32)

    segsum = _sc_segsum()
    xr, xroot = _proj1(x, w1c)
    parts1 = segsum(xr, src, dst, zrows)
    hr, hroot = _mid(parts1, xroot, b1r, w2c)
    parts2 = segsum(hr, src, dst, zrows)
    return _outk(parts2, hroot, b2r)


# speedup vs baseline: 20.9388x; 1.1530x over previous
import functools

import jax
import jax.numpy as jnp
from jax import lax
from jax.experimental import pallas as pl
from jax.experimental.pallas import tpu as pltpu
from jax.experimental.pallas import tpu_sc as plsc

N = 10000
D = 128
E = 320000
H = 16
C = 10

NC = 2
NS = 16
NW = NC * NS
CHUNK = 128
NCH = 80
EPW = NCH * CHUNK
E_PAD = NW * EPW
VROWS = 10240
ACC_ROWS = 10240
ZROWS = ACC_ROWS // NS
RPT = N // NS


G = 4
ROUNDS = NCH // G


def _sc_segsum_body(vals_hbm, src_hbm, dst_hbm, zeros_hbm, out_hbm,
                    src_v, dst_v, rows_v, acc_sh, gsA, gsB, ssA, ssB):
    c = lax.axis_index("c")
    s = lax.axis_index("s")
    wid = c * NS + s
    pltpu.sync_copy(zeros_hbm.at[pl.ds(s * ZROWS, ZROWS)],
                    acc_sh.at[pl.ds(s * ZROWS, ZROWS)])
    pltpu.sync_copy(src_hbm.at[wid], src_v)
    pltpu.sync_copy(dst_hbm.at[wid], dst_v)
    plsc.subcore_barrier()

    sems = (gsA, gsB, ssA, ssB)

    def start_gather(b, j):
        pltpu.async_copy(vals_hbm.at[src_v.at[j]], rows_v.at[b], sems[b])

    def wait_gather(b, j):
        pltpu.make_async_copy(vals_hbm.at[src_v.at[j]], rows_v.at[b],
                              sems[b]).wait()

    for b in range(G):
        start_gather(b, b)

    @pl.loop(0, NCH - G, step=G)
    def _(jj):
        for b in range(G):
            wait_gather(b, jj + b)
            pltpu.sync_copy(rows_v.at[b], acc_sh.at[dst_v.at[jj + b]],
                            add=True)
            start_gather(b, jj + G + b)

    for b in range(G):
        jj = NCH - G
        wait_gather(b, jj + b)
        pltpu.sync_copy(rows_v.at[b], acc_sh.at[dst_v.at[jj + b]], add=True)

    plsc.subcore_barrier()
    pltpu.sync_copy(acc_sh.at[pl.ds(s * ZROWS, ZROWS)],
                    out_hbm.at[c].at[pl.ds(s * ZROWS, ZROWS)])


@functools.cache
def _sc_segsum():
    mesh = plsc.VectorSubcoreMesh(core_axis_name="c", subcore_axis_name="s",
                                  num_cores=NC)
    return pl.kernel(
        _sc_segsum_body,
        out_type=jax.ShapeDtypeStruct((NC, ACC_ROWS, 16), jnp.float32),
        mesh=mesh,
        compiler_params=pltpu.CompilerParams(use_tc_tiling_on_sc=False),
        scratch_types=[
            pltpu.VMEM((NCH, CHUNK), jnp.int32),
            pltpu.VMEM((NCH, CHUNK), jnp.int32),
            pltpu.VMEM((G, CHUNK, 16), jnp.float32),
            pltpu.VMEM_SHARED((ACC_ROWS, 16), jnp.float32),
            pltpu.SemaphoreType.DMA,
            pltpu.SemaphoreType.DMA,
            pltpu.SemaphoreType.DMA,
            pltpu.SemaphoreType.DMA,
        ],
    )


def _proj1_body(x_ref, w_ref, xr_ref, xroot_ref):
    p = lax.dot_general(x_ref[...], w_ref[...], (((1,), (1,)), ((), ())),
                        preferred_element_type=jnp.float32)
    xr_ref[...] = jnp.concatenate(
        [p[:, :16], jnp.zeros((VROWS - N, 16), jnp.float32)], axis=0)
    xroot_ref[...] = p[:, 16:]


_proj1 = pl.pallas_call(
    _proj1_body,
    out_shape=(jax.ShapeDtypeStruct((VROWS, 16), jnp.float32),
               jax.ShapeDtypeStruct((N, 16), jnp.float32)),
)


def _mid_body(parts_ref, xroot_ref, b1_ref, w2_ref, hr_ref, hroot_ref):
    agg = parts_ref[0, :N] + parts_ref[1, :N]
    h = jnp.maximum(agg + xroot_ref[...] + b1_ref[...], 0.0)
    p = lax.dot_general(h, w2_ref[...], (((1,), (1,)), ((), ())),
                        preferred_element_type=jnp.float32)
    hr_ref[...] = jnp.concatenate(
        [p[:, :16], jnp.zeros((VROWS - N, 16), jnp.float32)], axis=0)
    hroot_ref[...] = p[:, 16:]


_mid = pl.pallas_call(
    _mid_body,
    out_shape=(jax.ShapeDtypeStruct((VROWS, 16), jnp.float32),
               jax.ShapeDtypeStruct((N, 16), jnp.float32)),
)


def _out_body(parts_ref, hroot_ref, b2_ref, o_ref):
    o = parts_ref[0, :N] + parts_ref[1, :N] + hroot_ref[...] + b2_ref[...]
    valid = lax.broadcasted_iota(jnp.int32, (1, 16), 1) < C
    om = jnp.where(valid, o, -1e30)
    m = jnp.max(om, axis=1, keepdims=True)
    lse = m + jnp.log(jnp.sum(jnp.exp(om - m), axis=1, keepdims=True))
    o_ref[...] = (o - lse)[:, :C]


_outk = pl.pallas_call(
    _out_body,
    out_shape=jax.ShapeDtypeStruct((N, C), jnp.float32),
)


def kernel(x, edge_index, W1_rel, b1, W1_root, W2_rel, b2, W2_root):
    w1c = jnp.concatenate([W1_rel, W1_root], axis=0)
    w2c = jnp.zeros((32, H), jnp.float32)
    w2c = w2c.at[:C].set(W2_rel).at[16:16 + C].set(W2_root)
    b1r = b1.reshape(1, H)
    b2r = jnp.zeros((1, 16), jnp.float32).at[:, :C].set(b2)

    ppw = EPW - E // NW
    pad = jnp.broadcast_to(N + jnp.arange(ppw, dtype=jnp.int32), (NW, ppw))
    src = jnp.concatenate([edge_index[0].reshape(NW, E // NW), pad], axis=1)
    dst = jnp.concatenate([edge_index[1].reshape(NW, E // NW), pad], axis=1)
    src = src.reshape(NW, NCH, CHUNK)
    dst = dst.reshape(NW, NCH, CHUNK)
    zrows = jnp.zeros((ACC_ROWS, 16), jnp.float32)

    segsum = _sc_segsum()
    xr, xroot = _proj1(x, w1c)
    parts1 = segsum(xr, src, dst, zrows)
    hr, hroot = _mid(parts1, xroot, b1r, w2c)
    parts2 = segsum(hr, src, dst, zrows)
    return _outk(parts2, hroot, b2r)


# speedup vs baseline: 23.0610x; 1.1014x over previous
import functools

import jax
import jax.numpy as jnp
from jax import lax
from jax.experimental import pallas as pl
from jax.experimental.pallas import tpu as pltpu
from jax.experimental.pallas import tpu_sc as plsc

N = 10000
D = 128
E = 320000
H = 16
C = 10

NC = 2
NS = 16
NW = NC * NS
CHUNK = 128
NCH = 80
EPW = NCH * CHUNK
E_PAD = NW * EPW
VROWS = 10240
ACC_ROWS = 10240
ZROWS = ACC_ROWS // NS
RPT = N // NS


G = 8
ROUNDS = NCH // G


def _sc_segsum_body(vals_hbm, src_hbm, dst_hbm, zeros_hbm, out_hbm,
                    src_v, dst_v, rows_v, acc_sh, *sems):
    c = lax.axis_index("c")
    s = lax.axis_index("s")
    wid = c * NS + s
    pltpu.sync_copy(zeros_hbm.at[pl.ds(s * ZROWS, ZROWS)],
                    acc_sh.at[pl.ds(s * ZROWS, ZROWS)])
    pltpu.sync_copy(src_hbm.at[wid], src_v)
    pltpu.sync_copy(dst_hbm.at[wid], dst_v)
    plsc.subcore_barrier()

    def start_gather(b, j):
        pltpu.async_copy(vals_hbm.at[src_v.at[j]], rows_v.at[b], sems[b])

    def wait_gather(b, j):
        pltpu.make_async_copy(vals_hbm.at[src_v.at[j]], rows_v.at[b],
                              sems[b]).wait()

    for b in range(G):
        start_gather(b, b)

    @pl.loop(0, NCH - G, step=G)
    def _(jj):
        for b in range(G):
            wait_gather(b, jj + b)
            pltpu.sync_copy(rows_v.at[b], acc_sh.at[dst_v.at[jj + b]],
                            add=True)
            start_gather(b, jj + G + b)

    for b in range(G):
        jj = NCH - G
        wait_gather(b, jj + b)
        pltpu.sync_copy(rows_v.at[b], acc_sh.at[dst_v.at[jj + b]], add=True)

    plsc.subcore_barrier()
    pltpu.sync_copy(acc_sh.at[pl.ds(s * ZROWS, ZROWS)],
                    out_hbm.at[c].at[pl.ds(s * ZROWS, ZROWS)])


@functools.cache
def _sc_segsum():
    mesh = plsc.VectorSubcoreMesh(core_axis_name="c", subcore_axis_name="s",
                                  num_cores=NC)
    return pl.kernel(
        _sc_segsum_body,
        out_type=jax.ShapeDtypeStruct((NC, ACC_ROWS, 16), jnp.float32),
        mesh=mesh,
        compiler_params=pltpu.CompilerParams(use_tc_tiling_on_sc=False),
        scratch_types=[
            pltpu.VMEM((NCH, CHUNK), jnp.int32),
            pltpu.VMEM((NCH, CHUNK), jnp.int32),
            pltpu.VMEM((G, CHUNK, 16), jnp.float32),
            pltpu.VMEM_SHARED((ACC_ROWS, 16), jnp.float32),
        ] + [pltpu.SemaphoreType.DMA] * G,
    )


def _proj1_body(x_ref, w_ref, xr_ref, xroot_ref):
    p = lax.dot_general(x_ref[...], w_ref[...], (((1,), (1,)), ((), ())),
                        preferred_element_type=jnp.float32)
    xr_ref[...] = jnp.concatenate(
        [p[:, :16], jnp.zeros((VROWS - N, 16), jnp.float32)], axis=0)
    xroot_ref[...] = p[:, 16:]


_proj1 = pl.pallas_call(
    _proj1_body,
    out_shape=(jax.ShapeDtypeStruct((VROWS, 16), jnp.float32),
               jax.ShapeDtypeStruct((N, 16), jnp.float32)),
)


def _mid_body(parts_ref, xroot_ref, b1_ref, w2_ref, hr_ref, hroot_ref):
    agg = parts_ref[0, :N] + parts_ref[1, :N]
    h = jnp.maximum(agg + xroot_ref[...] + b1_ref[...], 0.0)
    p = lax.dot_general(h, w2_ref[...], (((1,), (1,)), ((), ())),
                        preferred_element_type=jnp.float32)
    hr_ref[...] = jnp.concatenate(
        [p[:, :16], jnp.zeros((VROWS - N, 16), jnp.float32)], axis=0)
    hroot_ref[...] = p[:, 16:]


_mid = pl.pallas_call(
    _mid_body,
    out_shape=(jax.ShapeDtypeStruct((VROWS, 16), jnp.float32),
               jax.ShapeDtypeStruct((N, 16), jnp.float32)),
)


def _out_body(parts_ref, hroot_ref, b2_ref, o_ref):
    o = parts_ref[0, :N] + parts_ref[1, :N] + hroot_ref[...] + b2_ref[...]
    valid = lax.broadcasted_iota(jnp.int32, (1, 16), 1) < C
    om = jnp.where(valid, o, -1e30)
    m = jnp.max(om, axis=1, keepdims=True)
    lse = m + jnp.log(jnp.sum(jnp.exp(om - m), axis=1, keepdims=True))
    o_ref[...] = (o - lse)[:, :C]


_outk = pl.pallas_call(
    _out_body,
    out_shape=jax.ShapeDtypeStruct((N, C), jnp.float32),
)


def kernel(x, edge_index, W1_rel, b1, W1_root, W2_rel, b2, W2_root):
    w1c = jnp.concatenate([W1_rel, W1_root], axis=0)
    w2c = jnp.zeros((32, H), jnp.float32)
    w2c = w2c.at[:C].set(W2_rel).at[16:16 + C].set(W2_root)
    b1r = b1.reshape(1, H)
    b2r = jnp.zeros((1, 16), jnp.float32).at[:, :C].set(b2)

    ppw = EPW - E // NW
    pad = jnp.broadcast_to(N + jnp.arange(ppw, dtype=jnp.int32), (NW, ppw))
    src = jnp.concatenate([edge_index[0].reshape(NW, E // NW), pad], axis=1)
    dst = jnp.concatenate([edge_index[1].reshape(NW, E // NW), pad], axis=1)
    src = src.reshape(NW, NCH, CHUNK)
    dst = dst.reshape(NW, NCH, CHUNK)
    zrows = jnp.zeros((ACC_ROWS, 16), jnp.float32)

    segsum = _sc_segsum()
    xr, xroot = _proj1(x, w1c)
    parts1 = segsum(xr, src, dst, zrows)
    hr, hroot = _mid(parts1, xroot, b1r, w2c)
    parts2 = segsum(hr, src, dst, zrows)
    return _outk(parts2, hroot, b2r)
